# baseline (speedup 1.0000x reference)
import functools

import jax
import jax.numpy as jnp
from jax import lax
from jax.experimental import pallas as pl
from jax.experimental.pallas import tpu as pltpu
from jax.experimental.pallas import tpu_sc as plsc

N = 10000
E = 160000
D = 256
H = 8
C = 256
HC = H * C
LN = 16
AW = 128
NB = 40
NBLK = N // NB
NTILES = 32
BPT = (NBLK + NTILES - 1) // NTILES
CH = 3200
NCHUNK = E // CH
EB = 16
HS = 5

_GDN = lax.GatherDimensionNumbers(
    offset_dims=(), collapsed_slice_dims=(0,), start_index_map=(0,))


def _dyngather(v, idx):
    return lax.gather(v, idx[:, None], _GDN, (1,),
                      mode=lax.GatherScatterMode.PROMISE_IN_BOUNDS)


def _bcast_lane(v, j):
    return _dyngather(v, jnp.full((LN,), j, jnp.int32))


RB = 400


def _pre_body(x_ref, w_ref, aatt_ref, h_ref, asd_ref):
    h = jnp.dot(x_ref[...], w_ref[...], preferred_element_type=jnp.float32)
    h_ref[...] = h
    asd_ref[...] = jnp.dot(h, aatt_ref[...], preferred_element_type=jnp.float32)


def _pre(xin, W, aatt):
    return pl.pallas_call(
        _pre_body,
        grid=(N // RB,),
        in_specs=[
            pl.BlockSpec((RB, D), lambda i: (i, 0)),
            pl.BlockSpec((D, HC), lambda i: (0, 0)),
            pl.BlockSpec((HC, AW), lambda i: (0, 0)),
        ],
        out_specs=[
            pl.BlockSpec((RB, HC), lambda i: (i, 0)),
            pl.BlockSpec((RB, AW), lambda i: (i, 0)),
        ],
        out_shape=[
            jax.ShapeDtypeStruct((N, HC), jnp.float32),
            jax.ShapeDtypeStruct((N, AW), jnp.float32),
        ],
    )(xin, W, aatt)


_sc_mesh = plsc.VectorSubcoreMesh(core_axis_name="c", subcore_axis_name="s")


@functools.partial(
    pl.kernel,
    out_type=(
        jax.ShapeDtypeStruct((N, HC), jnp.float32),
        jax.ShapeDtypeStruct((N, LN), jnp.float32),
    ),
    mesh=_sc_mesh,
    compiler_params=pltpu.CompilerParams(needs_layout_passes=False),
    scratch_types=[
        pltpu.VMEM((CH,), jnp.int32),
        pltpu.VMEM((CH + LN,), jnp.int32),
        pltpu.VMEM((EB,), jnp.int32),
        pltpu.VMEM((EB,), jnp.int32),
        pltpu.VMEM((EB,), jnp.int32),
        pltpu.VMEM((EB, AW), jnp.float32),
        pltpu.VMEM((EB, AW), jnp.float32),
        pltpu.VMEM((EB, HC), jnp.float32),
        pltpu.VMEM((NB, HC), jnp.float32),
        pltpu.VMEM((NB, LN), jnp.float32),
    ],
)
def _sc_edge(h_hbm, asd_hbm, src_hbm, dst_hbm, S_hbm, den_hbm,
             dstbuf, elist, srcb, dstb, ldb, asd_s, asd_d, hrows, A, Aden):
    wid = lax.axis_index("c") * 16 + lax.axis_index("s")
    iota = lax.iota(jnp.int32, LN)
    shuf = (iota & 7) + 8

    for bi in range(BPT):
        blk = bi * NTILES + wid

        @pl.when(blk < NBLK)
        def _():
            base = blk * NB

            def zr(r, carry):
                def zc(c, c2):
                    A[r, pl.ds(c * LN, LN)] = jnp.zeros((LN,), jnp.float32)
                    return c2
                lax.fori_loop(0, HC // LN, zc, 0)
                Aden[r, :] = jnp.zeros((LN,), jnp.float32)
                return carry
            lax.fori_loop(0, NB, zr, 0)

            def chunk_body(k, carry):
                pltpu.sync_copy(dst_hbm.at[pl.ds(k * CH, CH)], dstbuf)

                def filt(i, cnt):
                    v = dstbuf[pl.ds(i * LN, LN)]
                    m = (v >= base) & (v < base + NB)
                    eid = k * CH + i * LN + iota
                    plsc.store_compressed(elist.at[pl.ds(cnt, LN)], eid, mask=m)
                    return cnt + jnp.sum(m.astype(jnp.int32))

                cnt = lax.fori_loop(0, CH // LN, filt, jnp.int32(0))
                elist[pl.ds(cnt, LN)] = jnp.zeros((LN,), jnp.int32)
                nbatch = (cnt + EB - 1) // EB

                def batch(bidx, bcarry):
                    ebref = elist.at[pl.ds(bidx * EB, EB)]
                    pltpu.sync_copy(src_hbm.at[ebref], srcb)
                    pltpu.sync_copy(dst_hbm.at[ebref], dstb)
                    pltpu.sync_copy(asd_hbm.at[srcb], asd_s)
                    pltpu.sync_copy(asd_hbm.at[dstb], asd_d)
                    pltpu.sync_copy(h_hbm.at[srcb], hrows)
                    dv = dstb[...]
                    ld_all = jnp.clip(dv - base, 0, NB - 1)

                    def edge(j, ecarry):
                        av = asd_s[j, pl.ds(0, LN)]
                        bv = asd_d[j, pl.ds(0, LN)]
                        z = av + _dyngather(bv, shuf)
                        z = jnp.where(z > 0.0, z, 0.2 * z)
                        w = jnp.exp(z)
                        w = jnp.where(bidx * EB + j < cnt, w,
                                      jnp.zeros((LN,), jnp.float32))
                        ldj = _bcast_lane(ld_all, j)
                        plsc.addupdate_scatter(Aden, [ldj, iota], w)
                        dsel = jnp.max(jnp.where(iota == j, dv,
                                                 jnp.zeros((LN,), jnp.int32)))
                        ld = jnp.clip(dsel - base, 0, NB - 1)

                        def head_s(hh, hcarry):
                            wh = _bcast_lane(w, hh)
                            for c in range(C // LN):
                                off = hh * C + c * LN
                                chunk = hrows[j, pl.ds(off, LN)]
                                plsc.addupdate_scatter(
                                    A, [ldj, off + iota], wh * chunk)
                            return hcarry

                        def head_d(hh, hcarry):
                            wh = _bcast_lane(w, hh)
                            for c in range(C // LN):
                                sl = pl.ds(hh * C + c * LN, LN)
                                A[ld, sl] = A[ld, sl] + wh * hrows[j, sl]
                            return hcarry

                        lax.fori_loop(0, HS, head_s, 0)
                        lax.fori_loop(HS, H, head_d, 0)
                        return ecarry

                    lax.fori_loop(0, EB, edge, 0)
                    return bcarry

                lax.fori_loop(0, nbatch, batch, 0)
                return carry

            lax.fori_loop(0, NCHUNK, chunk_body, 0)
            pltpu.sync_copy(A, S_hbm.at[pl.ds(base, NB)])
            pltpu.sync_copy(Aden, den_hbm.at[pl.ds(base, NB)])


def _post_body(S_ref, den_ref, as_ref, ad_ref, h_ref, b_ref, e8_ref, mn_ref,
               o_ref, *, final):
    z = as_ref[...] + ad_ref[...]
    z = jnp.where(z > 0.0, z, 0.2 * z)
    wself = jnp.exp(z)
    den8 = den_ref[...] + wself
    e8 = e8_ref[...]
    Sf = S_ref[...] + jnp.dot(wself, e8, preferred_element_type=jnp.float32) * h_ref[...]
    dfull = jnp.dot(den8, e8, preferred_element_type=jnp.float32) + 1e-16
    out = jnp.dot(Sf / dfull, mn_ref[...], preferred_element_type=jnp.float32)
    out = out + b_ref[...]
    if final:
        m = jnp.max(out, axis=1, keepdims=True)
        s = out - m
        o_ref[...] = s - jnp.log(jnp.sum(jnp.exp(s), axis=1, keepdims=True))
    else:
        o_ref[...] = jnp.maximum(out, 0.0)


def _post(S, den8, a_s, a_d, h2d, bias, e8, mn, final):
    return pl.pallas_call(
        functools.partial(_post_body, final=final),
        grid=(N // RB,),
        in_specs=[
            pl.BlockSpec((RB, HC), lambda i: (i, 0)),
            pl.BlockSpec((RB, H), lambda i: (i, 0)),
            pl.BlockSpec((RB, H), lambda i: (i, 0)),
            pl.BlockSpec((RB, H), lambda i: (i, 0)),
            pl.BlockSpec((RB, HC), lambda i: (i, 0)),
            pl.BlockSpec((1, C), lambda i: (0, 0)),
            pl.BlockSpec((H, HC), lambda i: (0, 0)),
            pl.BlockSpec((HC, C), lambda i: (0, 0)),
        ],
        out_specs=pl.BlockSpec((RB, C), lambda i: (i, 0)),
        out_shape=jax.ShapeDtypeStruct((N, C), jnp.float32),
    )(S, den8, a_s, a_d, h2d, bias, e8, mn)


def _aatt(att_src, att_dst):
    eye = jnp.eye(H, dtype=jnp.float32)
    ms = (att_src[0][:, :, None] * eye[:, None, :]).reshape(HC, H)
    md = (att_dst[0][:, :, None] * eye[:, None, :]).reshape(HC, H)
    pad = jnp.zeros((HC, AW - LN), jnp.float32)
    return jnp.concatenate([ms, md, pad], axis=1)


def _layer(xin, src, dst, W, att_src, att_dst, bias, e8, mn, final):
    h2d, asd = _pre(xin, W, _aatt(att_src, att_dst))
    S, den = _sc_edge(h2d, asd, src, dst)
    return _post(S, den[:, :H], asd[:, :H], asd[:, H:LN], h2d,
                 bias.reshape(1, C), e8, mn, final)


def kernel(x, edge_index, W1, att_src1, att_dst1, b1, W2, att_src2, att_dst2, b2):
    src = edge_index[0].astype(jnp.int32)
    dst = edge_index[1].astype(jnp.int32)
    e8 = jnp.kron(jnp.eye(H, dtype=jnp.float32), jnp.ones((1, C), jnp.float32))
    mn = jnp.kron(jnp.ones((H, 1), jnp.float32),
                  jnp.eye(C, dtype=jnp.float32)) / H
    h1 = _layer(x, src, dst, W1, att_src1, att_dst1, b1, e8, mn, final=False)
    return _layer(h1, src, dst, W2, att_src2, att_dst2, b2, e8, mn, final=True)


# speedup vs baseline: 1.0487x; 1.0487x over previous
import functools

import jax
import jax.numpy as jnp
from jax import lax
from jax.experimental import pallas as pl
from jax.experimental.pallas import tpu as pltpu
from jax.experimental.pallas import tpu_sc as plsc

N = 10000
E = 160000
D = 256
H = 8
C = 256
HC = H * C
LN = 16
AW = 128
NB = 40
NBLK = N // NB
NTILES = 32
BPT = (NBLK + NTILES - 1) // NTILES
CH = 3200
NCHUNK = E // CH
EB = 16

_GDN = lax.GatherDimensionNumbers(
    offset_dims=(), collapsed_slice_dims=(0,), start_index_map=(0,))


def _dyngather(v, idx):
    return lax.gather(v, idx[:, None], _GDN, (1,),
                      mode=lax.GatherScatterMode.PROMISE_IN_BOUNDS)


def _bcast_lane(v, j):
    return _dyngather(v, jnp.full((LN,), j, jnp.int32))


RB = 400


def _pre_body(x_ref, w_ref, aatt_ref, h_ref, asd_ref):
    h = jnp.dot(x_ref[...], w_ref[...], preferred_element_type=jnp.float32)
    h_ref[...] = h
    asd_ref[...] = jnp.dot(h, aatt_ref[...], preferred_element_type=jnp.float32)


def _pre(xin, W, aatt):
    return pl.pallas_call(
        _pre_body,
        grid=(N // RB,),
        in_specs=[
            pl.BlockSpec((RB, D), lambda i: (i, 0)),
            pl.BlockSpec((D, HC), lambda i: (0, 0)),
            pl.BlockSpec((HC, AW), lambda i: (0, 0)),
        ],
        out_specs=[
            pl.BlockSpec((RB, HC), lambda i: (i, 0)),
            pl.BlockSpec((RB, AW), lambda i: (i, 0)),
        ],
        out_shape=[
            jax.ShapeDtypeStruct((N, HC), jnp.float32),
            jax.ShapeDtypeStruct((N, AW), jnp.float32),
        ],
    )(xin, W, aatt)


_sc_mesh = plsc.VectorSubcoreMesh(core_axis_name="c", subcore_axis_name="s")


@functools.partial(
    pl.kernel,
    out_type=(
        jax.ShapeDtypeStruct((N, HC), jnp.float32),
        jax.ShapeDtypeStruct((N, LN), jnp.float32),
    ),
    mesh=_sc_mesh,
    compiler_params=pltpu.CompilerParams(needs_layout_passes=False),
    scratch_types=[
        pltpu.VMEM((CH,), jnp.int32),
        pltpu.VMEM((CH + LN,), jnp.int32),
        pltpu.VMEM((EB,), jnp.int32),
        pltpu.VMEM((EB,), jnp.int32),
        pltpu.VMEM((EB,), jnp.int32),
        pltpu.VMEM((EB, AW), jnp.float32),
        pltpu.VMEM((EB, AW), jnp.float32),
        pltpu.VMEM((EB, HC), jnp.float32),
        pltpu.VMEM((NB, HC), jnp.float32),
        pltpu.VMEM((NB, LN), jnp.float32),
    ],
)
def _sc_edge(h_hbm, asd_hbm, src_hbm, dst_hbm, S_hbm, den_hbm,
             dstbuf, elist, srcb, dstb, ldb, asd_s, asd_d, hrows, A, Aden):
    wid = lax.axis_index("c") * 16 + lax.axis_index("s")
    iota = lax.iota(jnp.int32, LN)
    shuf = (iota & 7) + 8

    for bi in range(BPT):
        blk = bi * NTILES + wid

        @pl.when(blk < NBLK)
        def _():
            base = blk * NB

            def zr(r, carry):
                def zc(c, c2):
                    A[r, pl.ds(c * LN, LN)] = jnp.zeros((LN,), jnp.float32)
                    return c2
                lax.fori_loop(0, HC // LN, zc, 0)
                Aden[r, :] = jnp.zeros((LN,), jnp.float32)
                return carry
            lax.fori_loop(0, NB, zr, 0)

            def chunk_body(k, carry):
                pltpu.sync_copy(dst_hbm.at[pl.ds(k * CH, CH)], dstbuf)

                def filt(i, cnt):
                    v = dstbuf[pl.ds(i * LN, LN)]
                    m = (v >= base) & (v < base + NB)
                    eid = k * CH + i * LN + iota
                    plsc.store_compressed(elist.at[pl.ds(cnt, LN)], eid, mask=m)
                    return cnt + jnp.sum(m.astype(jnp.int32))

                cnt = lax.fori_loop(0, CH // LN, filt, jnp.int32(0))
                elist[pl.ds(cnt, LN)] = jnp.zeros((LN,), jnp.int32)
                nbatch = (cnt + EB - 1) // EB

                def batch(bidx, bcarry):
                    ebref = elist.at[pl.ds(bidx * EB, EB)]
                    pltpu.sync_copy(src_hbm.at[ebref], srcb)
                    pltpu.sync_copy(dst_hbm.at[ebref], dstb)
                    pltpu.sync_copy(asd_hbm.at[srcb], asd_s)
                    pltpu.sync_copy(asd_hbm.at[dstb], asd_d)
                    pltpu.sync_copy(h_hbm.at[srcb], hrows)
                    dv = dstb[...]
                    ld_all = jnp.clip(dv - base, 0, NB - 1)

                    def edge(j, ecarry):
                        av = asd_s[j, pl.ds(0, LN)]
                        bv = asd_d[j, pl.ds(0, LN)]
                        z = av + _dyngather(bv, shuf)
                        z = jnp.where(z > 0.0, z, 0.2 * z)
                        w = jnp.exp(z)
                        w = jnp.where(bidx * EB + j < cnt, w,
                                      jnp.zeros((LN,), jnp.float32))
                        ldj = _bcast_lane(ld_all, j)
                        plsc.addupdate_scatter(Aden, [ldj, iota], w)
                        def head(hh, hcarry):
                            wh = _bcast_lane(w, hh)
                            for c in range(C // LN):
                                off = hh * C + c * LN
                                chunk = hrows[j, pl.ds(off, LN)]
                                plsc.addupdate_scatter(
                                    A, [ldj, off + iota], wh * chunk)
                            return hcarry

                        lax.fori_loop(0, H, head, 0)
                        return ecarry

                    lax.fori_loop(0, EB, edge, 0)
                    return bcarry

                lax.fori_loop(0, nbatch, batch, 0)
                return carry

            lax.fori_loop(0, NCHUNK, chunk_body, 0)
            pltpu.sync_copy(A, S_hbm.at[pl.ds(base, NB)])
            pltpu.sync_copy(Aden, den_hbm.at[pl.ds(base, NB)])


def _post_body(S_ref, den_ref, as_ref, ad_ref, h_ref, b_ref, e8_ref, mn_ref,
               o_ref, *, final):
    z = as_ref[...] + ad_ref[...]
    z = jnp.where(z > 0.0, z, 0.2 * z)
    wself = jnp.exp(z)
    den8 = den_ref[...] + wself
    e8 = e8_ref[...]
    Sf = S_ref[...] + jnp.dot(wself, e8, preferred_element_type=jnp.float32) * h_ref[...]
    dfull = jnp.dot(den8, e8, preferred_element_type=jnp.float32) + 1e-16
    out = jnp.dot(Sf / dfull, mn_ref[...], preferred_element_type=jnp.float32)
    out = out + b_ref[...]
    if final:
        m = jnp.max(out, axis=1, keepdims=True)
        s = out - m
        o_ref[...] = s - jnp.log(jnp.sum(jnp.exp(s), axis=1, keepdims=True))
    else:
        o_ref[...] = jnp.maximum(out, 0.0)


def _post(S, den8, a_s, a_d, h2d, bias, e8, mn, final):
    return pl.pallas_call(
        functools.partial(_post_body, final=final),
        grid=(N // RB,),
        in_specs=[
            pl.BlockSpec((RB, HC), lambda i: (i, 0)),
            pl.BlockSpec((RB, H), lambda i: (i, 0)),
            pl.BlockSpec((RB, H), lambda i: (i, 0)),
            pl.BlockSpec((RB, H), lambda i: (i, 0)),
            pl.BlockSpec((RB, HC), lambda i: (i, 0)),
            pl.BlockSpec((1, C), lambda i: (0, 0)),
            pl.BlockSpec((H, HC), lambda i: (0, 0)),
            pl.BlockSpec((HC, C), lambda i: (0, 0)),
        ],
        out_specs=pl.BlockSpec((RB, C), lambda i: (i, 0)),
        out_shape=jax.ShapeDtypeStruct((N, C), jnp.float32),
    )(S, den8, a_s, a_d, h2d, bias, e8, mn)


def _aatt(att_src, att_dst):
    eye = jnp.eye(H, dtype=jnp.float32)
    ms = (att_src[0][:, :, None] * eye[:, None, :]).reshape(HC, H)
    md = (att_dst[0][:, :, None] * eye[:, None, :]).reshape(HC, H)
    pad = jnp.zeros((HC, AW - LN), jnp.float32)
    return jnp.concatenate([ms, md, pad], axis=1)


def _layer(xin, src, dst, W, att_src, att_dst, bias, e8, mn, final):
    h2d, asd = _pre(xin, W, _aatt(att_src, att_dst))
    S, den = _sc_edge(h2d, asd, src, dst)
    return _post(S, den[:, :H], asd[:, :H], asd[:, H:LN], h2d,
                 bias.reshape(1, C), e8, mn, final)


def kernel(x, edge_index, W1, att_src1, att_dst1, b1, W2, att_src2, att_dst2, b2):
    src = edge_index[0].astype(jnp.int32)
    dst = edge_index[1].astype(jnp.int32)
    e8 = jnp.kron(jnp.eye(H, dtype=jnp.float32), jnp.ones((1, C), jnp.float32))
    mn = jnp.kron(jnp.ones((H, 1), jnp.float32),
                  jnp.eye(C, dtype=jnp.float32)) / H
    h1 = _layer(x, src, dst, W1, att_src1, att_dst1, b1, e8, mn, final=False)
    return _layer(h1, src, dst, W2, att_src2, att_dst2, b2, e8, mn, final=True)


# speedup vs baseline: 1.0488x; 1.0001x over previous
import functools

import jax
import jax.numpy as jnp
from jax import lax
from jax.experimental import pallas as pl
from jax.experimental.pallas import tpu as pltpu
from jax.experimental.pallas import tpu_sc as plsc

N = 10000
E = 160000
D = 256
H = 8
C = 256
HC = H * C
LN = 16
AW = 128
NB = 40
NBLK = N // NB
NTILES = 32
BPT = (NBLK + NTILES - 1) // NTILES
CH = 3200
NCHUNK = E // CH
EB = 16

_GDN = lax.GatherDimensionNumbers(
    offset_dims=(), collapsed_slice_dims=(0,), start_index_map=(0,))


def _dyngather(v, idx):
    return lax.gather(v, idx[:, None], _GDN, (1,),
                      mode=lax.GatherScatterMode.PROMISE_IN_BOUNDS)


def _bcast_lane(v, j):
    return _dyngather(v, jnp.full((LN,), j, jnp.int32))


RB = 400


def _pre_body(x_ref, w_ref, aatt_ref, h_ref, asd_ref):
    h = jnp.dot(x_ref[...], w_ref[...], preferred_element_type=jnp.float32)
    h_ref[...] = h
    asd_ref[...] = jnp.dot(h, aatt_ref[...], preferred_element_type=jnp.float32)


def _pre(xin, W, aatt):
    return pl.pallas_call(
        _pre_body,
        grid=(N // RB,),
        in_specs=[
            pl.BlockSpec((RB, D), lambda i: (i, 0)),
            pl.BlockSpec((D, HC), lambda i: (0, 0)),
            pl.BlockSpec((HC, AW), lambda i: (0, 0)),
        ],
        out_specs=[
            pl.BlockSpec((RB, HC), lambda i: (i, 0)),
            pl.BlockSpec((RB, AW), lambda i: (i, 0)),
        ],
        out_shape=[
            jax.ShapeDtypeStruct((N, HC), jnp.float32),
            jax.ShapeDtypeStruct((N, AW), jnp.float32),
        ],
    )(xin, W, aatt)


_sc_mesh = plsc.VectorSubcoreMesh(core_axis_name="c", subcore_axis_name="s")


@functools.partial(
    pl.kernel,
    out_type=(
        jax.ShapeDtypeStruct((N, HC), jnp.float32),
        jax.ShapeDtypeStruct((N, LN), jnp.float32),
    ),
    mesh=_sc_mesh,
    compiler_params=pltpu.CompilerParams(needs_layout_passes=False),
    scratch_types=[
        pltpu.VMEM((CH,), jnp.int32),
        pltpu.VMEM((CH + LN,), jnp.int32),
        pltpu.VMEM((EB,), jnp.int32),
        pltpu.VMEM((EB,), jnp.int32),
        pltpu.VMEM((EB, AW), jnp.float32),
        pltpu.VMEM((EB, AW), jnp.float32),
        pltpu.VMEM((EB, HC), jnp.float32),
        pltpu.VMEM((NB, HC), jnp.float32),
        pltpu.VMEM((NB, LN), jnp.float32),
    ],
)
def _sc_edge(h_hbm, asd_hbm, src_hbm, dst_hbm, S_hbm, den_hbm,
             dstbuf, elist, srcb, dstb, asd_s, asd_d, hrows, A, Aden):
    wid = lax.axis_index("c") * 16 + lax.axis_index("s")
    iota = lax.iota(jnp.int32, LN)
    shuf = (iota & 7) + 8

    for bi in range(BPT):
        blk = bi * NTILES + wid

        @pl.when(blk < NBLK)
        def _():
            base = blk * NB

            def zr(r, carry):
                def zc(c, c2):
                    A[r, pl.ds(c * LN, LN)] = jnp.zeros((LN,), jnp.float32)
                    return c2
                lax.fori_loop(0, HC // LN, zc, 0)
                Aden[r, :] = jnp.zeros((LN,), jnp.float32)
                return carry
            lax.fori_loop(0, NB, zr, 0)

            def chunk_body(k, carry):
                pltpu.sync_copy(dst_hbm.at[pl.ds(k * CH, CH)], dstbuf)

                def filt(i, cnt):
                    v = dstbuf[pl.ds(i * LN, LN)]
                    m = (v >= base) & (v < base + NB)
                    eid = k * CH + i * LN + iota
                    plsc.store_compressed(elist.at[pl.ds(cnt, LN)], eid, mask=m)
                    return cnt + jnp.sum(m.astype(jnp.int32))

                cnt = lax.fori_loop(0, CH // LN, filt, jnp.int32(0))
                elist[pl.ds(cnt, LN)] = jnp.zeros((LN,), jnp.int32)
                nbatch = (cnt + EB - 1) // EB

                def batch(bidx, bcarry):
                    ebref = elist.at[pl.ds(bidx * EB, EB)]
                    pltpu.sync_copy(src_hbm.at[ebref], srcb)
                    pltpu.sync_copy(dst_hbm.at[ebref], dstb)
                    pltpu.sync_copy(asd_hbm.at[srcb], asd_s)
                    pltpu.sync_copy(asd_hbm.at[dstb], asd_d)
                    pltpu.sync_copy(h_hbm.at[srcb], hrows)
                    dv = dstb[...]
                    ld_all = jnp.clip(dv - base, 0, NB - 1)

                    def edge(j, ecarry):
                        av = asd_s[j, pl.ds(0, LN)]
                        bv = asd_d[j, pl.ds(0, LN)]
                        z = av + _dyngather(bv, shuf)
                        z = jnp.where(z > 0.0, z, 0.2 * z)
                        w = jnp.exp(z)
                        w = jnp.where(bidx * EB + j < cnt, w,
                                      jnp.zeros((LN,), jnp.float32))
                        ldj = _bcast_lane(ld_all, j)
                        plsc.addupdate_scatter(Aden, [ldj, iota], w)
                        def head(hh, hcarry):
                            wh = _bcast_lane(w, hh)
                            for c in range(C // LN):
                                off = hh * C + c * LN
                                chunk = hrows[j, pl.ds(off, LN)]
                                plsc.addupdate_scatter(
                                    A, [ldj, off + iota], wh * chunk)
                            return hcarry

                        lax.fori_loop(0, H, head, 0)
                        return ecarry

                    lax.fori_loop(0, EB, edge, 0)
                    return bcarry

                lax.fori_loop(0, nbatch, batch, 0)
                return carry

            lax.fori_loop(0, NCHUNK, chunk_body, 0)
            pltpu.sync_copy(A, S_hbm.at[pl.ds(base, NB)])
            pltpu.sync_copy(Aden, den_hbm.at[pl.ds(base, NB)])


def _post_body(S_ref, den_ref, as_ref, ad_ref, h_ref, b_ref, e8_ref, mn_ref,
               o_ref, *, final):
    z = as_ref[...] + ad_ref[...]
    z = jnp.where(z > 0.0, z, 0.2 * z)
    wself = jnp.exp(z)
    den8 = den_ref[...] + wself
    e8 = e8_ref[...]
    Sf = S_ref[...] + jnp.dot(wself, e8, preferred_element_type=jnp.float32) * h_ref[...]
    dfull = jnp.dot(den8, e8, preferred_element_type=jnp.float32) + 1e-16
    out = jnp.dot(Sf / dfull, mn_ref[...], preferred_element_type=jnp.float32)
    out = out + b_ref[...]
    if final:
        m = jnp.max(out, axis=1, keepdims=True)
        s = out - m
        o_ref[...] = s - jnp.log(jnp.sum(jnp.exp(s), axis=1, keepdims=True))
    else:
        o_ref[...] = jnp.maximum(out, 0.0)


def _post(S, den8, a_s, a_d, h2d, bias, e8, mn, final):
    return pl.pallas_call(
        functools.partial(_post_body, final=final),
        grid=(N // RB,),
        in_specs=[
            pl.BlockSpec((RB, HC), lambda i: (i, 0)),
            pl.BlockSpec((RB, H), lambda i: (i, 0)),
            pl.BlockSpec((RB, H), lambda i: (i, 0)),
            pl.BlockSpec((RB, H), lambda i: (i, 0)),
            pl.BlockSpec((RB, HC), lambda i: (i, 0)),
            pl.BlockSpec((1, C), lambda i: (0, 0)),
            pl.BlockSpec((H, HC), lambda i: (0, 0)),
            pl.BlockSpec((HC, C), lambda i: (0, 0)),
        ],
        out_specs=pl.BlockSpec((RB, C), lambda i: (i, 0)),
        out_shape=jax.ShapeDtypeStruct((N, C), jnp.float32),
    )(S, den8, a_s, a_d, h2d, bias, e8, mn)


def _aatt(att_src, att_dst):
    eye = jnp.eye(H, dtype=jnp.float32)
    ms = (att_src[0][:, :, None] * eye[:, None, :]).reshape(HC, H)
    md = (att_dst[0][:, :, None] * eye[:, None, :]).reshape(HC, H)
    pad = jnp.zeros((HC, AW - LN), jnp.float32)
    return jnp.concatenate([ms, md, pad], axis=1)


def _layer(xin, src, dst, W, att_src, att_dst, bias, e8, mn, final):
    h2d, asd = _pre(xin, W, _aatt(att_src, att_dst))
    S, den = _sc_edge(h2d, asd, src, dst)
    return _post(S, den[:, :H], asd[:, :H], asd[:, H:LN], h2d,
                 bias.reshape(1, C), e8, mn, final)


def kernel(x, edge_index, W1, att_src1, att_dst1, b1, W2, att_src2, att_dst2, b2):
    src = edge_index[0].astype(jnp.int32)
    dst = edge_index[1].astype(jnp.int32)
    e8 = jnp.kron(jnp.eye(H, dtype=jnp.float32), jnp.ones((1, C), jnp.float32))
    mn = jnp.kron(jnp.ones((H, 1), jnp.float32),
                  jnp.eye(C, dtype=jnp.float32)) / H
    h1 = _layer(x, src, dst, W1, att_src1, att_dst1, b1, e8, mn, final=False)
    return _layer(h1, src, dst, W2, att_src2, att_dst2, b2, e8, mn, final=True)
